# baseline (speedup 1.0000x reference)
import functools
import math

import jax
import jax.numpy as jnp
from jax import lax
from jax.experimental import pallas as pl
from jax.experimental.pallas import tpu as pltpu
from jax.experimental.pallas import tpu_sc as plsc

H = 12
Q = 2048
K = 2048
NB = 32
TW = 4224
NSHIFT = 8
HP = 16
_LOG_RATIO = math.log(128 / 8)

NW = 32
T128W = 4096
NG = Q // 128
T128ROWS = H * 128
RPW128 = T128ROWS // NW


def _table_kernel(s_ref, wt_ref, out_ref):
    t = pl.program_id(0)
    d = lax.broadcasted_iota(jnp.int32, (1, TW), 1) + (t - (Q - 1) + s_ref[0])
    ret = (d >= 0).astype(jnp.int32) * (NB // 2)
    n = jnp.abs(d)
    max_exact = NB // 4
    n_safe = jnp.maximum(n, 1)
    val_if_large = max_exact + (
        jnp.log(n_safe.astype(jnp.float32) / max_exact)
        / _LOG_RATIO
        * (NB // 2 - max_exact)
    ).astype(jnp.int32)
    val_if_large = jnp.minimum(val_if_large, NB // 2 - 1)
    bucket = ret + jnp.where(n < max_exact, n, val_if_large)
    b_iota = lax.broadcasted_iota(jnp.int32, (NB, TW), 0)
    onehot = (bucket == b_iota).astype(jnp.float32)
    out_ref[...] = jnp.dot(
        wt_ref[...], onehot,
        preferred_element_type=jnp.float32,
        precision=lax.Precision.HIGHEST,
    )


def _build_table(s, weight_t):
    return pl.pallas_call(
        _table_kernel,
        grid=(NSHIFT,),
        in_specs=[
            pl.BlockSpec(memory_space=pltpu.SMEM),
            pl.BlockSpec((HP, NB), lambda t: (0, 0)),
        ],
        out_specs=pl.BlockSpec((HP, TW), lambda t: (t, 0)),
        out_shape=jax.ShapeDtypeStruct((NSHIFT * HP, TW), jnp.float32),
    )(s, weight_t)


NH_A = 4
NH_B = 8
CHUNK = 16


def _t128_body(h0, nh, table_hbm, out_hbm, buf, sem_r, sem_w):
    rpw = nh * 128 // NW
    wid = lax.axis_index("s") * 2 + lax.axis_index("c")
    r0 = wid * rpw

    for chunk in range(rpw // CHUNK):
        base = r0 + chunk * CHUNK

        def fire_read(jj, carry):
            rr = base + jj
            h_local = rr // 128
            k = rr - h_local * 128
            h = h0 + h_local
            shift = 127 - k
            b = lax.rem(shift, NSHIFT)
            q8 = shift - b
            src_off = pl.multiple_of((b * HP + h) * TW + q8, NSHIFT)
            pltpu.make_async_copy(
                table_hbm.at[pl.ds(src_off, T128W)],
                buf.at[pl.ds(jj * T128W, T128W)],
                sem_r,
            ).start()
            return carry

        lax.fori_loop(0, CHUNK, fire_read, 0)

        def drain_read(jj, carry):
            pltpu.make_async_copy(
                table_hbm.at[pl.ds(0, T128W)],
                buf.at[pl.ds(0, T128W)],
                sem_r,
            ).wait()
            return carry

        lax.fori_loop(0, CHUNK, drain_read, 0)

        def fire_write(jj, carry):
            rr = base + jj
            pltpu.make_async_copy(
                buf.at[pl.ds(jj * T128W, T128W)],
                out_hbm.at[pl.ds(rr * T128W, T128W)],
                sem_w,
            ).start()
            return carry

        lax.fori_loop(0, CHUNK, fire_write, 0)

        def drain_write(jj, carry):
            pltpu.make_async_copy(
                buf.at[pl.ds(0, T128W)],
                out_hbm.at[pl.ds(0, T128W)],
                sem_w,
            ).wait()
            return carry

        lax.fori_loop(0, CHUNK, drain_write, 0)


def _build_t128_half(table_flat, h0, nh):
    kern = pl.kernel(
        functools.partial(_t128_body, h0, nh),
        out_type=jax.ShapeDtypeStruct((nh * 128 * T128W,), jnp.float32),
        mesh=plsc.VectorSubcoreMesh(core_axis_name="c", subcore_axis_name="s"),
        scratch_types=[
            pltpu.VMEM((CHUNK * T128W,), jnp.float32),
            pltpu.SemaphoreType.DMA,
            pltpu.SemaphoreType.DMA,
        ],
    )
    return kern(table_flat).reshape(nh, 128, T128W)


def _expand_kernel_first(t128_ref, out_ref):
    for g in range(NG):
        c0 = (NG - 1 - g) * 128
        out_ref[0, 0, g * 128:(g + 1) * 128, :] = t128_ref[0, :, c0:c0 + K]


def _expand_kernel_second(t128_ref, prev_ref, out_ref):
    del prev_ref
    for g in range(NG):
        c0 = (NG - 1 - g) * 128
        out_ref[0, 0, g * 128:(g + 1) * 128, :] = t128_ref[0, :, c0:c0 + K]


def _expand_half(t128_half, h0, nh, prev=None):
    out_shape = jax.ShapeDtypeStruct((1, H, Q, K), jnp.float32)
    in_specs = [pl.BlockSpec((1, 128, T128W), lambda h: (h, 0, 0))]
    operands = [t128_half]
    body = _expand_kernel_first
    aliases = {}
    if prev is not None:
        in_specs.append(pl.BlockSpec(memory_space=pltpu.HBM))
        operands.append(prev)
        body = _expand_kernel_second
        aliases = {1: 0}
    return pl.pallas_call(
        body,
        grid=(nh,),
        in_specs=in_specs,
        out_specs=pl.BlockSpec((1, 1, Q, K), lambda h, h0=h0: (0, h0 + h, 0, 0)),
        out_shape=out_shape,
        input_output_aliases=aliases,
    )(*operands)


def kernel(num_queries, num_keys, weight):
    s = (jnp.asarray(num_queries, jnp.int32) - jnp.int32(Q)).reshape(1)
    weight_t = jnp.zeros((HP, NB), jnp.float32).at[:H, :].set(weight.T)
    table_flat = _build_table(s, weight_t).reshape(NSHIFT * HP * TW)
    t128_a = _build_t128_half(table_flat, 0, NH_A)
    t128_b = _build_t128_half(table_flat, NH_A, NH_B)
    out = _expand_half(t128_a, 0, NH_A)
    out = _expand_half(t128_b, NH_A, NH_B, prev=out)
    return out


# speedup vs baseline: 1.0557x; 1.0557x over previous
import functools
import math

import jax
import jax.numpy as jnp
from jax import lax
from jax.experimental import pallas as pl
from jax.experimental.pallas import tpu as pltpu
from jax.experimental.pallas import tpu_sc as plsc

H = 12
Q = 2048
K = 2048
NB = 32
TW = 4224
NSHIFT = 8
HP = 16
_LOG_RATIO = math.log(128 / 8)

NW = 32
T128W = 4096
NG = Q // 128
T128ROWS = H * 128
RPW128 = T128ROWS // NW


def _table_kernel(s_ref, wt_ref, out_ref):
    t = pl.program_id(0)
    d = lax.broadcasted_iota(jnp.int32, (1, TW), 1) + (t - (Q - 1) + s_ref[0])
    ret = (d >= 0).astype(jnp.int32) * (NB // 2)
    n = jnp.abs(d)
    max_exact = NB // 4
    n_safe = jnp.maximum(n, 1)
    val_if_large = max_exact + (
        jnp.log(n_safe.astype(jnp.float32) / max_exact)
        / _LOG_RATIO
        * (NB // 2 - max_exact)
    ).astype(jnp.int32)
    val_if_large = jnp.minimum(val_if_large, NB // 2 - 1)
    bucket = ret + jnp.where(n < max_exact, n, val_if_large)
    b_iota = lax.broadcasted_iota(jnp.int32, (NB, TW), 0)
    onehot = (bucket == b_iota).astype(jnp.float32)
    out_ref[...] = jnp.dot(
        wt_ref[...], onehot,
        preferred_element_type=jnp.float32,
        precision=lax.Precision.HIGHEST,
    )


def _build_table(s, weight_t):
    return pl.pallas_call(
        _table_kernel,
        grid=(NSHIFT,),
        in_specs=[
            pl.BlockSpec(memory_space=pltpu.SMEM),
            pl.BlockSpec((HP, NB), lambda t: (0, 0)),
        ],
        out_specs=pl.BlockSpec((HP, TW), lambda t: (t, 0)),
        out_shape=jax.ShapeDtypeStruct((NSHIFT * HP, TW), jnp.float32),
    )(s, weight_t)


NH_A = 6
NH_B = 6
CHUNK = 16


def _t128_body(h0, nh, table_hbm, out_hbm, buf, sem_r, sem_w):
    rpw = nh * 128 // NW
    wid = lax.axis_index("s") * 2 + lax.axis_index("c")
    r0 = wid * rpw

    for chunk in range(rpw // CHUNK):
        base = r0 + chunk * CHUNK

        def fire_read(jj, carry):
            rr = base + jj
            h_local = rr // 128
            k = rr - h_local * 128
            h = h0 + h_local
            shift = 127 - k
            b = lax.rem(shift, NSHIFT)
            q8 = shift - b
            src_off = pl.multiple_of((b * HP + h) * TW + q8, NSHIFT)
            pltpu.make_async_copy(
                table_hbm.at[pl.ds(src_off, T128W)],
                buf.at[pl.ds(jj * T128W, T128W)],
                sem_r,
            ).start()
            return carry

        lax.fori_loop(0, CHUNK, fire_read, 0)

        def drain_read(jj, carry):
            pltpu.make_async_copy(
                table_hbm.at[pl.ds(0, T128W)],
                buf.at[pl.ds(0, T128W)],
                sem_r,
            ).wait()
            return carry

        lax.fori_loop(0, CHUNK, drain_read, 0)

        def fire_write(jj, carry):
            rr = base + jj
            pltpu.make_async_copy(
                buf.at[pl.ds(jj * T128W, T128W)],
                out_hbm.at[pl.ds(rr * T128W, T128W)],
                sem_w,
            ).start()
            return carry

        lax.fori_loop(0, CHUNK, fire_write, 0)

        def drain_write(jj, carry):
            pltpu.make_async_copy(
                buf.at[pl.ds(0, T128W)],
                out_hbm.at[pl.ds(0, T128W)],
                sem_w,
            ).wait()
            return carry

        lax.fori_loop(0, CHUNK, drain_write, 0)


def _build_t128_half(table_flat, h0, nh):
    kern = pl.kernel(
        functools.partial(_t128_body, h0, nh),
        out_type=jax.ShapeDtypeStruct((nh * 128 * T128W,), jnp.float32),
        mesh=plsc.VectorSubcoreMesh(core_axis_name="c", subcore_axis_name="s"),
        scratch_types=[
            pltpu.VMEM((CHUNK * T128W,), jnp.float32),
            pltpu.SemaphoreType.DMA,
            pltpu.SemaphoreType.DMA,
        ],
    )
    return kern(table_flat).reshape(nh, 128, T128W)


def _expand_kernel_first(t128_ref, out_ref):
    for g in range(NG):
        c0 = (NG - 1 - g) * 128
        out_ref[0, 0, g * 128:(g + 1) * 128, :] = t128_ref[0, :, c0:c0 + K]


def _expand_kernel_second(t128_ref, prev_ref, out_ref):
    del prev_ref
    for g in range(NG):
        c0 = (NG - 1 - g) * 128
        out_ref[0, 0, g * 128:(g + 1) * 128, :] = t128_ref[0, :, c0:c0 + K]


def _expand_half(t128_half, h0, nh, prev=None):
    out_shape = jax.ShapeDtypeStruct((1, H, Q, K), jnp.float32)
    in_specs = [pl.BlockSpec((1, 128, T128W), lambda h: (h, 0, 0))]
    operands = [t128_half]
    body = _expand_kernel_first
    aliases = {}
    if prev is not None:
        in_specs.append(pl.BlockSpec(memory_space=pltpu.HBM))
        operands.append(prev)
        body = _expand_kernel_second
        aliases = {1: 0}
    return pl.pallas_call(
        body,
        grid=(nh,),
        in_specs=in_specs,
        out_specs=pl.BlockSpec((1, 1, Q, K), lambda h, h0=h0: (0, h0 + h, 0, 0)),
        out_shape=out_shape,
        input_output_aliases=aliases,
    )(*operands)


def kernel(num_queries, num_keys, weight):
    s = (jnp.asarray(num_queries, jnp.int32) - jnp.int32(Q)).reshape(1)
    weight_t = jnp.zeros((HP, NB), jnp.float32).at[:H, :].set(weight.T)
    table_flat = _build_table(s, weight_t).reshape(NSHIFT * HP * TW)
    t128_a = _build_t128_half(table_flat, 0, NH_A)
    t128_b = _build_t128_half(table_flat, NH_A, NH_B)
    out = _expand_half(t128_a, 0, NH_A)
    out = _expand_half(t128_b, NH_A, NH_B, prev=out)
    return out
